# Initial kernel scaffold; baseline (speedup 1.0000x reference)
import functools

import jax
import jax.numpy as jnp
from jax import lax
from jax.experimental import pallas as pl
from jax.experimental.pallas import tpu as pltpu
from jax.experimental.pallas import tpu_sc as plsc

_N = 10000
_E = 320000
_D = 128
_G = 64

_NC = 2
_NS = 16
_L = 16
_NW = _NC * _NS

_C = 80
_ROWS_PER_SUB = _N // _NS
_STAGE = 125
_N_STAGE = _ROWS_PER_SUB // _STAGE


def _make_sc_agg(n_nodes, n_edges, d):
    e_per_w = n_edges // _NW
    n_chunks = e_per_w // _C
    mesh = plsc.VectorSubcoreMesh(core_axis_name="c", subcore_axis_name="s")

    @functools.partial(
        pl.kernel,
        mesh=mesh,
        out_type=jax.ShapeDtypeStruct((_NC, n_nodes, d), jnp.float32),
        scratch_types=[
            pltpu.VMEM((_C,), jnp.int32),
            pltpu.VMEM((_C,), jnp.int32),
            pltpu.VMEM((_C, d), jnp.float32),
            pltpu.VMEM((_STAGE, d), jnp.float32),
            pltpu.VMEM_SHARED((n_nodes, d), jnp.float32),
            pltpu.SemaphoreType.DMA,
        ],
    )
    def agg(x_hbm, src_hbm, dst_hbm, out_hbm,
            src_v, dst_v, rows_v, stage_v, acc_sh, sem):
        cid = lax.axis_index("c")
        sid = lax.axis_index("s")
        wid = cid * _NS + sid
        row0 = sid * _ROWS_PER_SUB

        def zrow(r, carry):
            for c8 in range(d // _L):
                stage_v[r, pl.ds(c8 * _L, _L)] = jnp.zeros((_L,), jnp.float32)
            return carry

        lax.fori_loop(0, _STAGE, zrow, 0)

        def zcopy(j, carry):
            pltpu.sync_copy(stage_v, acc_sh.at[pl.ds(row0 + j * _STAGE, _STAGE)])
            return carry

        lax.fori_loop(0, _N_STAGE, zcopy, 0)
        plsc.subcore_barrier()

        base0 = wid * e_per_w

        def body(i, carry):
            base = base0 + i * _C
            pltpu.sync_copy(src_hbm.at[pl.ds(base, _C)], src_v)
            pltpu.sync_copy(dst_hbm.at[pl.ds(base, _C)], dst_v)
            pltpu.async_copy(x_hbm.at[src_v], rows_v, sem).wait()
            pltpu.sync_copy(rows_v, acc_sh.at[dst_v], add=True)
            return carry

        lax.fori_loop(0, n_chunks, body, 0)
        plsc.subcore_barrier()

        def out_body(j, carry):
            r = row0 + j * _STAGE
            pltpu.sync_copy(acc_sh.at[pl.ds(r, _STAGE)], stage_v)
            pltpu.sync_copy(stage_v, out_hbm.at[cid, pl.ds(r, _STAGE)])
            return carry

        lax.fori_loop(0, _N_STAGE, out_body, 0)

    return agg


_sc_agg = _make_sc_agg(_N, _E, _D)


def _mlp(x, Wa, ba, g, be, Wb, bb):
    h = jnp.dot(x, Wa, preferred_element_type=jnp.float32) + ba
    m = jnp.mean(h, axis=0, keepdims=True)
    v = jnp.mean((h - m) ** 2, axis=0, keepdims=True)
    h = (h - m) * lax.rsqrt(v + 1e-5) * g + be
    h = jnp.maximum(h, 0.0)
    h = jnp.dot(h, Wb, preferred_element_type=jnp.float32) + bb
    return jnp.maximum(h, 0.0)


def _mlp_kernel(x_ref, agg_ref, Wa_ref, ba_ref, g_ref, be_ref, Wb_ref,
                bb_ref, out_ref):
    x = x_ref[...] + agg_ref[0] + agg_ref[1]
    out_ref[...] = _mlp(x, Wa_ref[...], ba_ref[...], g_ref[...],
                        be_ref[...], Wb_ref[...], bb_ref[...])


def _mlp_pool_kernel(x_ref, agg_ref, batch_ref, Wa_ref, ba_ref, g_ref,
                     be_ref, Wb_ref, bb_ref, Wl_ref, bl_ref, out_ref):
    x = x_ref[...] + agg_ref[0] + agg_ref[1]
    h = _mlp(x, Wa_ref[...], ba_ref[...], g_ref[...], be_ref[...],
             Wb_ref[...], bb_ref[...])
    onehot = (batch_ref[...] ==
              lax.broadcasted_iota(jnp.int32, (_G, _N), 0)).astype(jnp.float32)
    pooled = jnp.dot(onehot, h, preferred_element_type=jnp.float32)
    out_ref[...] = (jnp.dot(pooled, Wl_ref[...],
                            preferred_element_type=jnp.float32) + bl_ref[...])


def kernel(x, edge_index, batch, W1a, b1a, g1, be1, W1b, b1b,
           W2a, b2a, g2, be2, W2b, b2b, Wl, bl):
    src = edge_index[0]
    dst = edge_index[1]

    agg1 = _sc_agg(x, src, dst)
    h1 = pl.pallas_call(
        _mlp_kernel,
        out_shape=jax.ShapeDtypeStruct((_N, _D), jnp.float32),
    )(x, agg1, W1a, b1a.reshape(1, -1), g1.reshape(1, -1),
      be1.reshape(1, -1), W1b, b1b.reshape(1, -1))

    agg2 = _sc_agg(h1, src, dst)
    out = pl.pallas_call(
        _mlp_pool_kernel,
        out_shape=jax.ShapeDtypeStruct((_G, 1), jnp.float32),
    )(h1, agg2, batch.reshape(1, -1), W2a, b2a.reshape(1, -1),
      g2.reshape(1, -1), be2.reshape(1, -1), W2b, b2b.reshape(1, -1),
      Wl, bl.reshape(1, -1))
    return out


# speedup vs baseline: 4.8349x; 4.8349x over previous
import functools

import jax
import jax.numpy as jnp
from jax import lax
from jax.experimental import pallas as pl
from jax.experimental.pallas import tpu as pltpu
from jax.experimental.pallas import tpu_sc as plsc

_N = 10000
_E = 320000
_D = 128
_G = 64

_NC = 2
_NS = 16
_L = 16
_NW = _NC * _NS

_C = 80
_NP = 10240
_ROWS_PER_SUB = _NP // _NS
_STAGE = 128
_N_STAGE = _ROWS_PER_SUB // _STAGE


def _make_sc_agg(n_edges, d):
    e_per_w = n_edges // _NW
    n_chunks = e_per_w // _C
    mesh = plsc.VectorSubcoreMesh(core_axis_name="c", subcore_axis_name="s")

    @functools.partial(
        pl.kernel,
        mesh=mesh,
        out_type=jax.ShapeDtypeStruct((_NC, _NP, d), jnp.float32),
        scratch_types=[
            pltpu.VMEM((_C,), jnp.int32),
            pltpu.VMEM((_C,), jnp.int32),
            pltpu.VMEM((_C, d), jnp.float32),
            pltpu.VMEM((_STAGE, d), jnp.float32),
            pltpu.VMEM_SHARED((_NP, d), jnp.float32),
            pltpu.SemaphoreType.DMA,
        ],
    )
    def agg(x_hbm, src_hbm, dst_hbm, out_hbm,
            src_v, dst_v, rows_v, stage_v, acc_sh, sem):
        cid = lax.axis_index("c")
        sid = lax.axis_index("s")
        wid = cid * _NS + sid
        row0 = sid * _ROWS_PER_SUB

        def zrow(r, carry):
            for c8 in range(d // _L):
                stage_v[r, pl.ds(c8 * _L, _L)] = jnp.zeros((_L,), jnp.float32)
            return carry

        lax.fori_loop(0, _STAGE, zrow, 0)

        def zcopy(j, carry):
            pltpu.sync_copy(stage_v, acc_sh.at[pl.ds(row0 + j * _STAGE, _STAGE)])
            return carry

        lax.fori_loop(0, _N_STAGE, zcopy, 0)
        plsc.subcore_barrier()

        base0 = wid * e_per_w

        def body(i, carry):
            base = base0 + i * _C
            pltpu.sync_copy(src_hbm.at[pl.ds(base, _C)], src_v)
            pltpu.sync_copy(dst_hbm.at[pl.ds(base, _C)], dst_v)
            pltpu.async_copy(x_hbm.at[src_v], rows_v, sem).wait()
            pltpu.sync_copy(rows_v, acc_sh.at[dst_v], add=True)
            return carry

        lax.fori_loop(0, n_chunks, body, 0)
        plsc.subcore_barrier()

        def out_body(j, carry):
            r = row0 + j * _STAGE
            pltpu.sync_copy(acc_sh.at[pl.ds(r, _STAGE)], stage_v)
            pltpu.sync_copy(stage_v, out_hbm.at[cid, pl.ds(r, _STAGE)])
            return carry

        lax.fori_loop(0, _N_STAGE, out_body, 0)

    return agg


_sc_agg = _make_sc_agg(_E, _D)


def _mlp(x, Wa, ba, g, be, Wb, bb):
    h = jnp.dot(x, Wa, preferred_element_type=jnp.float32) + ba
    m = jnp.mean(h, axis=0, keepdims=True)
    v = jnp.mean((h - m) ** 2, axis=0, keepdims=True)
    h = (h - m) * lax.rsqrt(v + 1e-5) * g + be
    h = jnp.maximum(h, 0.0)
    h = jnp.dot(h, Wb, preferred_element_type=jnp.float32) + bb
    return jnp.maximum(h, 0.0)


def _mlp_kernel(x_ref, agg_ref, Wa_ref, ba_ref, g_ref, be_ref, Wb_ref,
                bb_ref, out_ref):
    x = x_ref[...] + agg_ref[0, :_N, :] + agg_ref[1, :_N, :]
    out_ref[...] = _mlp(x, Wa_ref[...], ba_ref[...], g_ref[...],
                        be_ref[...], Wb_ref[...], bb_ref[...])


def _mlp_pool_kernel(x_ref, agg_ref, batch_ref, Wa_ref, ba_ref, g_ref,
                     be_ref, Wb_ref, bb_ref, Wl_ref, bl_ref, out_ref):
    x = x_ref[...] + agg_ref[0, :_N, :] + agg_ref[1, :_N, :]
    h = _mlp(x, Wa_ref[...], ba_ref[...], g_ref[...], be_ref[...],
             Wb_ref[...], bb_ref[...])
    onehot = (batch_ref[...] ==
              lax.broadcasted_iota(jnp.int32, (_G, _N), 0)).astype(jnp.float32)
    pooled = jnp.dot(onehot, h, preferred_element_type=jnp.float32)
    out_ref[...] = (jnp.dot(pooled, Wl_ref[...],
                            preferred_element_type=jnp.float32) + bl_ref[...])


def kernel(x, edge_index, batch, W1a, b1a, g1, be1, W1b, b1b,
           W2a, b2a, g2, be2, W2b, b2b, Wl, bl):
    src = edge_index[0]
    dst = edge_index[1]

    agg1 = _sc_agg(x, src, dst)
    h1 = pl.pallas_call(
        _mlp_kernel,
        out_shape=jax.ShapeDtypeStruct((_N, _D), jnp.float32),
    )(x, agg1, W1a, b1a.reshape(1, -1), g1.reshape(1, -1),
      be1.reshape(1, -1), W1b, b1b.reshape(1, -1))

    agg2 = _sc_agg(h1, src, dst)
    out = pl.pallas_call(
        _mlp_pool_kernel,
        out_shape=jax.ShapeDtypeStruct((_G, 1), jnp.float32),
    )(h1, agg2, batch.reshape(1, -1), W2a, b2a.reshape(1, -1),
      g2.reshape(1, -1), be2.reshape(1, -1), W2b, b2b.reshape(1, -1),
      Wl, bl.reshape(1, -1))
    return out


# speedup vs baseline: 11.1478x; 2.3057x over previous
import functools

import jax
import jax.numpy as jnp
from jax import lax
from jax.experimental import pallas as pl
from jax.experimental.pallas import tpu as pltpu
from jax.experimental.pallas import tpu_sc as plsc

_N = 10000
_E = 320000
_D = 128
_G = 64

_NC = 2
_NS = 16
_L = 16
_NW = _NC * _NS

_C = 80
_NCHUNK = _E // (_NW * _C)
_NP = 10240
_ROWS_PER_SUB = _NP // _NS
_N_STAGE = _ROWS_PER_SUB // _C


def _make_sc_agg(d):
    mesh = plsc.VectorSubcoreMesh(core_axis_name="c", subcore_axis_name="s")

    @functools.partial(
        pl.kernel,
        mesh=mesh,
        out_type=jax.ShapeDtypeStruct((_NC, _NP, d), jnp.float32),
        scratch_types=[
            pltpu.VMEM((_NCHUNK * _C,), jnp.int32),
            pltpu.VMEM((_NCHUNK * _C,), jnp.int32),
            pltpu.VMEM((_C,), jnp.int32),
            pltpu.VMEM((_C, d), jnp.float32),
            pltpu.VMEM((_C, d), jnp.float32),
            pltpu.VMEM_SHARED((_NP, d), jnp.float32),
            pltpu.SemaphoreType.DMA,
            pltpu.SemaphoreType.DMA,
        ],
    )
    def agg(x_hbm, src_hbm, dst_hbm, out_hbm,
            src_v, dst_v, didx, buf0, buf1, acc_sh, sem0, sem1):
        cid = lax.axis_index("c")
        sid = lax.axis_index("s")
        wid = cid * _NS + sid
        row0 = sid * _ROWS_PER_SUB
        e_per_w = _NCHUNK * _C

        pltpu.sync_copy(src_hbm.at[pl.ds(wid * e_per_w, e_per_w)], src_v)
        pltpu.sync_copy(dst_hbm.at[pl.ds(wid * e_per_w, e_per_w)], dst_v)

        def zrow(r, carry):
            for c8 in range(d // _L):
                buf0[r, pl.ds(c8 * _L, _L)] = jnp.zeros((_L,), jnp.float32)
            return carry

        lax.fori_loop(0, _C, zrow, 0)

        def zcopy(j, carry):
            pltpu.sync_copy(buf0, acc_sh.at[pl.ds(row0 + j * _C, _C)])
            return carry

        lax.fori_loop(0, _N_STAGE, zcopy, 0)
        plsc.subcore_barrier()

        bufs = (buf0, buf1)
        sems = (sem0, sem1)
        n_pairs = _NCHUNK // 2

        def gather(p, b):
            pltpu.async_copy(
                x_hbm.at[src_v.at[pl.ds(p * _C, _C)]], bufs[b], sems[b])

        def wait_gather(b):
            pltpu.make_async_copy(x_hbm.at[pl.ds(0, _C)], bufs[b], sems[b]).wait()

        def scatter(p, b):
            for k in range(_C // _L):
                didx[pl.ds(k * _L, _L)] = dst_v[pl.ds(p * _C + k * _L, _L)]
            pltpu.sync_copy(bufs[b], acc_sh.at[didx], add=True)

        gather(0, 0)

        def body(q, carry):
            p0 = 2 * q
            gather(p0 + 1, 1)
            wait_gather(0)
            scatter(p0, 0)
            gather(p0 + 2, 0)
            wait_gather(1)
            scatter(p0 + 1, 1)
            return carry

        lax.fori_loop(0, n_pairs, body, 0)
        wait_gather(0)
        scatter(_NCHUNK - 1, 0)
        plsc.subcore_barrier()

        def out_body(j, carry):
            r = row0 + j * _C
            pltpu.sync_copy(acc_sh.at[pl.ds(r, _C)], buf0)
            pltpu.sync_copy(buf0, out_hbm.at[cid, pl.ds(r, _C)])
            return carry

        lax.fori_loop(0, _N_STAGE, out_body, 0)

    return agg


_sc_agg = _make_sc_agg(_D)


def _mlp(x, Wa, ba, g, be, Wb, bb):
    h = jnp.dot(x, Wa, preferred_element_type=jnp.float32) + ba
    m = jnp.mean(h, axis=0, keepdims=True)
    v = jnp.mean((h - m) ** 2, axis=0, keepdims=True)
    h = (h - m) * lax.rsqrt(v + 1e-5) * g + be
    h = jnp.maximum(h, 0.0)
    h = jnp.dot(h, Wb, preferred_element_type=jnp.float32) + bb
    return jnp.maximum(h, 0.0)


def _mlp_kernel(x_ref, agg_ref, Wa_ref, ba_ref, g_ref, be_ref, Wb_ref,
                bb_ref, out_ref):
    x = x_ref[...] + agg_ref[0, :_N, :] + agg_ref[1, :_N, :]
    out_ref[...] = _mlp(x, Wa_ref[...], ba_ref[...], g_ref[...],
                        be_ref[...], Wb_ref[...], bb_ref[...])


def _mlp_pool_kernel(x_ref, agg_ref, batch_ref, Wa_ref, ba_ref, g_ref,
                     be_ref, Wb_ref, bb_ref, Wl_ref, bl_ref, out_ref):
    x = x_ref[...] + agg_ref[0, :_N, :] + agg_ref[1, :_N, :]
    h = _mlp(x, Wa_ref[...], ba_ref[...], g_ref[...], be_ref[...],
             Wb_ref[...], bb_ref[...])
    onehot = (batch_ref[...] ==
              lax.broadcasted_iota(jnp.int32, (_G, _N), 0)).astype(jnp.float32)
    pooled = jnp.dot(onehot, h, preferred_element_type=jnp.float32)
    out_ref[...] = (jnp.dot(pooled, Wl_ref[...],
                            preferred_element_type=jnp.float32) + bl_ref[...])


def kernel(x, edge_index, batch, W1a, b1a, g1, be1, W1b, b1b,
           W2a, b2a, g2, be2, W2b, b2b, Wl, bl):
    src = edge_index[0]
    dst = edge_index[1]

    agg1 = _sc_agg(x, src, dst)
    h1 = pl.pallas_call(
        _mlp_kernel,
        out_shape=jax.ShapeDtypeStruct((_N, _D), jnp.float32),
    )(x, agg1, W1a, b1a.reshape(1, -1), g1.reshape(1, -1),
      be1.reshape(1, -1), W1b, b1b.reshape(1, -1))

    agg2 = _sc_agg(h1, src, dst)
    out = pl.pallas_call(
        _mlp_pool_kernel,
        out_shape=jax.ShapeDtypeStruct((_G, 1), jnp.float32),
    )(h1, agg2, batch.reshape(1, -1), W2a, b2a.reshape(1, -1),
      g2.reshape(1, -1), be2.reshape(1, -1), W2b, b2b.reshape(1, -1),
      Wl, bl.reshape(1, -1))
    return out
